# baseline (speedup 1.0000x reference)
import math

import jax
import jax.numpy as jnp
from jax import lax
from jax.experimental import pallas as pl
from jax.experimental.pallas import tpu as pltpu
from jax.experimental.pallas import tpu_sc as plsc

_SIGMA = 0.1
_N = 4096
_L = 16
_NC = 2
_NS = 16
_NW = _NC * _NS
_AG = 4

_K = 2048
_KB = _K // _L

_BND = tuple(round(_KB * math.sqrt(w / _NW)) for w in range(_NW + 1))

_BR = 256
_CW = 256
_TC_PROGS = (_N - _K) // _BR


def _bnd_arr():
    return jnp.array(_BND + (0,) * (3 * _L - len(_BND)), jnp.int32)


def _bcast_lane(vec, idxv):
    return lax.gather(
        vec,
        idxv[:, None],
        lax.GatherDimensionNumbers(
            offset_dims=(), collapsed_slice_dims=(0,), start_index_map=(0,)
        ),
        (1,),
        indices_are_sorted=False,
        unique_indices=False,
        mode=lax.GatherScatterMode.PROMISE_IN_BOUNDS,
    )


def _cindex_sc_kernel(t_hbm, e_hbm, f_hbm, bnd_hbm, out_hbm, t_v, e_v, f_v, bnd_v, o_v):
    wid = lax.axis_index("s") * _NC + lax.axis_index("c")
    pltpu.sync_copy(t_hbm, t_v)
    pltpu.sync_copy(e_hbm, e_v)
    pltpu.sync_copy(f_hbm, f_v)
    pltpu.sync_copy(bnd_hbm, bnd_v)

    lo_blk = bnd_v[pl.ds(wid, 1)][0]
    hi_blk = bnd_v[pl.ds(wid + 1, 1)][0]

    def exp_body(i, c):
        sl = pl.ds(i * _L, _L)
        e_v[sl] = jnp.exp(e_v[sl] * (1.0 / _SIGMA))
        return c

    lax.fori_loop(0, hi_blk, exp_body, 0)

    zero = jnp.zeros((_L,), jnp.float32)

    def a_body(bi, carry):
        sl_a = pl.ds(bi * _L, _L)
        ta_blk = t_v[sl_a]
        ea_blk = e_v[sl_a]

        def k_body(kg, carry2):
            splats = []
            for j in range(_AG):
                idxv = jnp.full((_L,), kg * _AG + j, jnp.int32)
                splats.append((_bcast_lane(ta_blk, idxv), _bcast_lane(ea_blk, idxv)))

            def b_body(c, carry3):
                accn3, accd3 = carry3
                sl = pl.ds(c * _L, _L)
                tb = t_v[sl]
                eb = e_v[sl]
                fb = f_v[sl]
                for ta, ea in splats:
                    mf = jnp.where(ta > tb, fb, 0.0)
                    q = ea / (ea + eb)
                    accn3 = accn3 + q * mf
                    accd3 = accd3 + mf
                return accn3, accd3

            return lax.fori_loop(0, bi + 1, b_body, carry2)

        return lax.fori_loop(0, _L // _AG, k_body, carry)

    accn, accd = lax.fori_loop(lo_blk, hi_blk, a_body, (zero, zero))
    o_v[pl.ds(0, _L)] = accn
    o_v[pl.ds(_L, _L)] = accd
    pltpu.sync_copy(o_v, out_hbm.at[wid])


def _cindex_tc_kernel(tcol_ref, scol_ref, trow_ref, srow_ref, frow_ref, out_ref):
    i = pl.program_id(0)
    ta = tcol_ref[...]
    ea = jnp.exp(scol_ref[...] * (1.0 / _SIGMA))
    ncols = _K // _CW + i + 1

    def c_body(c, carry):
        accn, accd = carry
        sl = pl.ds(c * _CW, _CW)
        tb = trow_ref[:, sl]
        eb = jnp.exp(srow_ref[:, sl] * (1.0 / _SIGMA))
        fb = frow_ref[:, sl]
        mf = jnp.where(ta > tb, fb, 0.0)
        q = ea / (ea + eb)
        return accn + jnp.sum(q * mf), accd + jnp.sum(mf)

    num, den = lax.fori_loop(0, ncols, c_body, (0.0, 0.0))
    out_ref[...] = jnp.stack([num, den]).reshape(1, 1, 2)


@jax.jit
def kernel(times, scores, events):
    order = jnp.argsort(times)
    ts = times[order]
    ss = scores[order]
    fs = events[order].astype(jnp.float32)

    mesh = plsc.VectorSubcoreMesh(core_axis_name="c", subcore_axis_name="s")
    sc_partials = pl.kernel(
        _cindex_sc_kernel,
        mesh=mesh,
        out_type=jax.ShapeDtypeStruct((_NW, 2 * _L), jnp.float32),
        scratch_types=[
            pltpu.VMEM((_K,), jnp.float32),
            pltpu.VMEM((_K,), jnp.float32),
            pltpu.VMEM((_K,), jnp.float32),
            pltpu.VMEM((3 * _L,), jnp.int32),
            pltpu.VMEM((2 * _L,), jnp.float32),
        ],
    )(ts[:_K], ss[:_K], fs[:_K], _bnd_arr())

    tcol = ts[_K:].reshape(-1, 1)
    scol = ss[_K:].reshape(-1, 1)
    trow = ts.reshape(1, _N)
    srow = ss.reshape(1, _N)
    frow = fs.reshape(1, _N)

    tc_partials = pl.pallas_call(
        _cindex_tc_kernel,
        grid=(_TC_PROGS,),
        in_specs=[
            pl.BlockSpec((_BR, 1), lambda i: (i, 0)),
            pl.BlockSpec((_BR, 1), lambda i: (i, 0)),
            pl.BlockSpec((1, _N), lambda i: (0, 0)),
            pl.BlockSpec((1, _N), lambda i: (0, 0)),
            pl.BlockSpec((1, _N), lambda i: (0, 0)),
        ],
        out_specs=pl.BlockSpec((1, 1, 2), lambda i: (i, 0, 0)),
        out_shape=jax.ShapeDtypeStruct((_TC_PROGS, 1, 2), jnp.float32),
        compiler_params=pltpu.CompilerParams(
            dimension_semantics=("arbitrary",),
        ),
    )(tcol, scol, trow, srow, frow)

    num = sc_partials[:, :_L].sum() + tc_partials[:, 0, 0].sum()
    den = sc_partials[:, _L:].sum() + tc_partials[:, 0, 1].sum()
    return num / (den + 1.0)


# speedup vs baseline: 1.2409x; 1.2409x over previous
import math

import jax
import jax.numpy as jnp
from jax import lax
from jax.experimental import pallas as pl
from jax.experimental.pallas import tpu as pltpu
from jax.experimental.pallas import tpu_sc as plsc

_SIGMA = 0.1
_N = 4096
_L = 16
_NC = 2
_NS = 16
_NW = _NC * _NS
_AG = 4

_K = 2048
_KB = _K // _L

_BND = tuple(round(_KB * math.sqrt(w / _NW)) for w in range(_NW + 1))

_BR = 256
_CW = 256
_TC_PROGS = (_N - _K) // _BR


def _bnd_arr():
    return jnp.array(_BND + (0,) * (3 * _L - len(_BND)), jnp.int32)


def _bcast_lane(vec, idxv):
    return lax.gather(
        vec,
        idxv[:, None],
        lax.GatherDimensionNumbers(
            offset_dims=(), collapsed_slice_dims=(0,), start_index_map=(0,)
        ),
        (1,),
        indices_are_sorted=False,
        unique_indices=False,
        mode=lax.GatherScatterMode.PROMISE_IN_BOUNDS,
    )


def _cindex_sc_kernel(t_hbm, e_hbm, f_hbm, bnd_hbm, out_hbm, t_v, e_v, f_v, bnd_v, o_v):
    wid = lax.axis_index("s") * _NC + lax.axis_index("c")
    pltpu.sync_copy(t_hbm, t_v)
    pltpu.sync_copy(e_hbm, e_v)
    pltpu.sync_copy(f_hbm, f_v)
    pltpu.sync_copy(bnd_hbm, bnd_v)

    lo_blk = bnd_v[pl.ds(wid, 1)][0]
    hi_blk = bnd_v[pl.ds(wid + 1, 1)][0]

    def exp_body(i, c):
        sl = pl.ds(i * _L, _L)
        e_v[sl] = jnp.exp(e_v[sl] * (1.0 / _SIGMA))
        return c

    lax.fori_loop(0, hi_blk, exp_body, 0)

    zero = jnp.zeros((_L,), jnp.float32)

    def a_body(bi, carry):
        sl_a = pl.ds(bi * _L, _L)
        ta_blk = t_v[sl_a]
        ea_blk = e_v[sl_a]

        def k_body(kg, carry2):
            splats = []
            for j in range(_AG):
                idxv = jnp.full((_L,), kg * _AG + j, jnp.int32)
                splats.append((_bcast_lane(ta_blk, idxv), _bcast_lane(ea_blk, idxv)))

            def b_body(c, carry3):
                accn3, accd3 = carry3
                sl = pl.ds(c * _L, _L)
                tb = t_v[sl]
                eb = e_v[sl]
                fb = f_v[sl]
                for ta, ea in splats:
                    mf = jnp.where(ta > tb, fb, 0.0)
                    q = ea / (ea + eb)
                    accn3 = accn3 + q * mf
                    accd3 = accd3 + mf
                return accn3, accd3

            return lax.fori_loop(0, bi + 1, b_body, carry2)

        return lax.fori_loop(0, _L // _AG, k_body, carry)

    accn, accd = lax.fori_loop(lo_blk, hi_blk, a_body, (zero, zero))
    o_v[pl.ds(0, _L)] = accn
    o_v[pl.ds(_L, _L)] = accd
    pltpu.sync_copy(o_v, out_hbm.at[wid])


def _cindex_tc_kernel(tcol_ref, scol_ref, trow_ref, srow_ref, frow_ref, out_ref):
    i = pl.program_id(0)
    ta = tcol_ref[...]
    ea = jnp.exp(scol_ref[...] * (1.0 / _SIGMA))
    ncols = _K // _CW + i + 1

    def c_body(c, carry):
        accn, accd = carry
        sl = pl.ds(c * _CW, _CW)
        tb = trow_ref[:, sl]
        eb = jnp.exp(srow_ref[:, sl] * (1.0 / _SIGMA))
        fb = frow_ref[:, sl]
        mf = jnp.where(ta > tb, fb, 0.0)
        q = ea / (ea + eb)
        return accn + jnp.sum(q * mf), accd + jnp.sum(mf)

    num, den = lax.fori_loop(0, ncols, c_body, (0.0, 0.0))
    out_ref[...] = jnp.stack([num, den]).reshape(1, 1, 2)


@jax.jit
def kernel(times, scores, events):
    ts, ss, fs = lax.sort(
        (times, scores, events.astype(jnp.float32)), num_keys=1
    )

    mesh = plsc.VectorSubcoreMesh(core_axis_name="c", subcore_axis_name="s")
    sc_partials = pl.kernel(
        _cindex_sc_kernel,
        mesh=mesh,
        out_type=jax.ShapeDtypeStruct((_NW, 2 * _L), jnp.float32),
        scratch_types=[
            pltpu.VMEM((_K,), jnp.float32),
            pltpu.VMEM((_K,), jnp.float32),
            pltpu.VMEM((_K,), jnp.float32),
            pltpu.VMEM((3 * _L,), jnp.int32),
            pltpu.VMEM((2 * _L,), jnp.float32),
        ],
    )(ts[:_K], ss[:_K], fs[:_K], _bnd_arr())

    tcol = ts[_K:].reshape(-1, 1)
    scol = ss[_K:].reshape(-1, 1)
    trow = ts.reshape(1, _N)
    srow = ss.reshape(1, _N)
    frow = fs.reshape(1, _N)

    tc_partials = pl.pallas_call(
        _cindex_tc_kernel,
        grid=(_TC_PROGS,),
        in_specs=[
            pl.BlockSpec((_BR, 1), lambda i: (i, 0)),
            pl.BlockSpec((_BR, 1), lambda i: (i, 0)),
            pl.BlockSpec((1, _N), lambda i: (0, 0)),
            pl.BlockSpec((1, _N), lambda i: (0, 0)),
            pl.BlockSpec((1, _N), lambda i: (0, 0)),
        ],
        out_specs=pl.BlockSpec((1, 1, 2), lambda i: (i, 0, 0)),
        out_shape=jax.ShapeDtypeStruct((_TC_PROGS, 1, 2), jnp.float32),
        compiler_params=pltpu.CompilerParams(
            dimension_semantics=("arbitrary",),
        ),
    )(tcol, scol, trow, srow, frow)

    num = sc_partials[:, :_L].sum() + tc_partials[:, 0, 0].sum()
    den = sc_partials[:, _L:].sum() + tc_partials[:, 0, 1].sum()
    return num / (den + 1.0)


# speedup vs baseline: 1.2418x; 1.0007x over previous
import math

import jax
import jax.numpy as jnp
from jax import lax
from jax.experimental import pallas as pl
from jax.experimental.pallas import tpu as pltpu
from jax.experimental.pallas import tpu_sc as plsc

_SIGMA = 0.1
_N = 4096
_L = 16
_NC = 2
_NS = 16
_NW = _NC * _NS
_AG = 4

_K = 2048
_KB = _K // _L

_BND = tuple(round(_KB * math.sqrt(w / _NW)) for w in range(_NW + 1))

_BR = 256
_CW = 256
_TC_PROGS = (_N - _K) // _BR


def _bnd_arr():
    return jnp.array(_BND + (0,) * (3 * _L - len(_BND)), jnp.int32)


def _bcast_lane(vec, idxv):
    return lax.gather(
        vec,
        idxv[:, None],
        lax.GatherDimensionNumbers(
            offset_dims=(), collapsed_slice_dims=(0,), start_index_map=(0,)
        ),
        (1,),
        indices_are_sorted=False,
        unique_indices=False,
        mode=lax.GatherScatterMode.PROMISE_IN_BOUNDS,
    )


def _cindex_sc_kernel(t_hbm, e_hbm, f_hbm, bnd_hbm, out_hbm, t_v, e_v, f_v, bnd_v, o_v):
    wid = lax.axis_index("s") * _NC + lax.axis_index("c")
    pltpu.sync_copy(t_hbm, t_v)
    pltpu.sync_copy(e_hbm, e_v)
    pltpu.sync_copy(f_hbm, f_v)
    pltpu.sync_copy(bnd_hbm, bnd_v)

    lo_blk = bnd_v[pl.ds(wid, 1)][0]
    hi_blk = bnd_v[pl.ds(wid + 1, 1)][0]

    def exp_body(i, c):
        sl = pl.ds(i * _L, _L)
        e_v[sl] = jnp.exp(e_v[sl] * (1.0 / _SIGMA))
        return c

    lax.fori_loop(0, hi_blk, exp_body, 0)

    zero = jnp.zeros((_L,), jnp.float32)

    def a_body(bi, carry):
        sl_a = pl.ds(bi * _L, _L)
        ta_blk = t_v[sl_a]
        ea_blk = e_v[sl_a]

        def k_body(kg, carry2):
            splats = []
            for j in range(_AG):
                idxv = jnp.full((_L,), kg * _AG + j, jnp.int32)
                splats.append((_bcast_lane(ta_blk, idxv), _bcast_lane(ea_blk, idxv)))

            def b_body(g, carry3):
                accn3, accd3 = carry3
                for u in range(4):
                    sl = pl.ds((g * 4 + u) * _L, _L)
                    tb = t_v[sl]
                    eb = e_v[sl]
                    fb = f_v[sl]
                    for ta, ea in splats:
                        mf = jnp.where(ta > tb, fb, 0.0)
                        q = ea / (ea + eb)
                        accn3 = accn3 + q * mf
                        accd3 = accd3 + mf
                return accn3, accd3

            return lax.fori_loop(0, (bi + 4) // 4, b_body, carry2)

        return lax.fori_loop(0, _L // _AG, k_body, carry)

    accn, accd = lax.fori_loop(lo_blk, hi_blk, a_body, (zero, zero))
    o_v[pl.ds(0, _L)] = accn
    o_v[pl.ds(_L, _L)] = accd
    pltpu.sync_copy(o_v, out_hbm.at[wid])


def _cindex_tc_kernel(tcol_ref, scol_ref, trow_ref, srow_ref, frow_ref, out_ref):
    i = pl.program_id(0)
    ta = tcol_ref[...]
    ea = jnp.exp(scol_ref[...] * (1.0 / _SIGMA))
    ncols = _K // _CW + i + 1

    def c_body(c, carry):
        accn, accd = carry
        sl = pl.ds(c * _CW, _CW)
        tb = trow_ref[:, sl]
        eb = jnp.exp(srow_ref[:, sl] * (1.0 / _SIGMA))
        fb = frow_ref[:, sl]
        mf = jnp.where(ta > tb, fb, 0.0)
        q = ea / (ea + eb)
        return accn + jnp.sum(q * mf), accd + jnp.sum(mf)

    num, den = lax.fori_loop(0, ncols, c_body, (0.0, 0.0))
    out_ref[...] = jnp.stack([num, den]).reshape(1, 1, 2)


@jax.jit
def kernel(times, scores, events):
    ts, ss, fs = lax.sort(
        (times, scores, events.astype(jnp.float32)), num_keys=1
    )

    mesh = plsc.VectorSubcoreMesh(core_axis_name="c", subcore_axis_name="s")
    sc_partials = pl.kernel(
        _cindex_sc_kernel,
        mesh=mesh,
        out_type=jax.ShapeDtypeStruct((_NW, 2 * _L), jnp.float32),
        scratch_types=[
            pltpu.VMEM((_K,), jnp.float32),
            pltpu.VMEM((_K,), jnp.float32),
            pltpu.VMEM((_K,), jnp.float32),
            pltpu.VMEM((3 * _L,), jnp.int32),
            pltpu.VMEM((2 * _L,), jnp.float32),
        ],
    )(ts[:_K], ss[:_K], fs[:_K], _bnd_arr())

    tcol = ts[_K:].reshape(-1, 1)
    scol = ss[_K:].reshape(-1, 1)
    trow = ts.reshape(1, _N)
    srow = ss.reshape(1, _N)
    frow = fs.reshape(1, _N)

    tc_partials = pl.pallas_call(
        _cindex_tc_kernel,
        grid=(_TC_PROGS,),
        in_specs=[
            pl.BlockSpec((_BR, 1), lambda i: (i, 0)),
            pl.BlockSpec((_BR, 1), lambda i: (i, 0)),
            pl.BlockSpec((1, _N), lambda i: (0, 0)),
            pl.BlockSpec((1, _N), lambda i: (0, 0)),
            pl.BlockSpec((1, _N), lambda i: (0, 0)),
        ],
        out_specs=pl.BlockSpec((1, 1, 2), lambda i: (i, 0, 0)),
        out_shape=jax.ShapeDtypeStruct((_TC_PROGS, 1, 2), jnp.float32),
        compiler_params=pltpu.CompilerParams(
            dimension_semantics=("arbitrary",),
        ),
    )(tcol, scol, trow, srow, frow)

    num = sc_partials[:, :_L].sum() + tc_partials[:, 0, 0].sum()
    den = sc_partials[:, _L:].sum() + tc_partials[:, 0, 1].sum()
    return num / (den + 1.0)
